# baseline (speedup 1.0000x reference)
import functools

import jax
import jax.numpy as jnp
from jax import lax
from jax.experimental import pallas as pl
from jax.experimental.pallas import tpu as pltpu
from jax.experimental.pallas import tpu_sc as plsc

_N, _D, _C, _E = 10000, 128, 16, 320000


_NC, _NS = 2, 16
_NW = _NC * _NS
_EPW = _E // _NW
_CH = 80
_FULL = 3 * (-(-(-(-_EPW // _CH)) // 3))
_PAD = _FULL * _CH - _EPW
_NDUMMY = _NS
_RPT = 624
_RTAIL = _N - _RPT * _NS
_KMAX = _FULL // 3

@functools.cache
def _make_sc_agg():
    mesh = plsc.VectorSubcoreMesh(core_axis_name="c", subcore_axis_name="s")
    return pl.kernel(
        _sc_agg_body,
        mesh=mesh,
        out_type=jax.ShapeDtypeStruct((_NC, _N, _D), jnp.float32),
        scratch_types=(
            [pltpu.VMEM((_FULL, _CH), jnp.int32)]
            + [pltpu.VMEM((_CH,), jnp.int32) for _ in range(3)]
            + [pltpu.VMEM((_CH, _D), jnp.float32) for _ in range(3)]
            + [pltpu.VMEM_SHARED((_N + _NDUMMY, _D), jnp.float32)]
            + [pltpu.SemaphoreType.DMA for _ in range(9)]
        ),
    )


def _sc_agg(x, edges, zeros):
    return _make_sc_agg()(x, *edges, zeros)


def _split_edges(edge_index):
    src = edge_index[0].reshape(_NW, _EPW)
    dst = edge_index[1].reshape(_NW, _EPW)
    srcp = jnp.concatenate(
        [src, jnp.zeros((_NW, _PAD), jnp.int32)], axis=1)
    dpad = jnp.broadcast_to(
        (_N + (jnp.arange(_NW) // _NC)).astype(jnp.int32)[:, None],
        (_NW, _PAD))
    dstp = jnp.concatenate([dst, dpad], axis=1)
    return (srcp.reshape(_NW, _FULL, _CH), dstp.reshape(_NW, _FULL, _CH))


def _sc_agg_body(x_hbm, src_hbm, dst_hbm, zero_hbm, out_hbm, *refs):
    src_v = refs[0]
    db = refs[1:4]
    rows = refs[4:7]
    agg_sh = refs[7]
    sd = refs[8:11]
    sg = refs[11:14]
    ss = refs[14:17]
    cid = lax.axis_index("c")
    sid = lax.axis_index("s")
    wid = sid * _NC + cid

    pltpu.sync_copy(zero_hbm.at[pl.ds(0, _RPT)],
                    agg_sh.at[pl.ds(sid * _RPT, _RPT)])

    @pl.when(sid == _NS - 1)
    def _():
        pltpu.sync_copy(zero_hbm.at[pl.ds(0, _RTAIL)],
                        agg_sh.at[pl.ds(_NS * _RPT, _RTAIL)])

    pltpu.sync_copy(src_hbm.at[wid], src_v)
    plsc.subcore_barrier()

    def _dst_load(b, i):
        pltpu.async_copy(dst_hbm.at[wid, i], db[b], sd[b])

    def _wait_dst(b, i):
        pltpu.make_async_copy(dst_hbm.at[wid, i], db[b], sd[b]).wait()

    def _gather(b, i):
        pltpu.async_copy(x_hbm.at[src_v.at[i]], rows[b], sg[b])

    def _wait_gather(b, i):
        pltpu.make_async_copy(x_hbm.at[src_v.at[i]], rows[b], sg[b]).wait()

    def _scatter(b):
        pltpu.async_copy(rows[b], agg_sh.at[db[b]], ss[b], add=True)

    def _wait_scatter(b):
        pltpu.make_async_copy(rows[b], agg_sh.at[db[b]], ss[b]).wait()

    _dst_load(0, 0)
    _dst_load(1, 1)
    _gather(0, 0)
    _gather(1, 1)

    def body(k, carry):
        i_base = 3 * k
        for t in range(3):
            i = i_base + t
            r = t
            r2 = (t + 2) % 3
            _wait_gather(r, i)
            _wait_dst(r, i)
            _scatter(r)
            if t == 0:
                @pl.when(k > 0)
                def _():
                    _wait_scatter(r2)
            else:
                _wait_scatter(r2)
            if t == 0:
                _dst_load(r2, i + 2)
                _gather(r2, i + 2)
            else:
                @pl.when(k < _KMAX - 1)
                def _():
                    _dst_load(r2, i + 2)
                    _gather(r2, i + 2)
        return carry

    lax.fori_loop(0, _KMAX, body, 0)
    _wait_scatter((_FULL - 1) % 3)


_R = 1000
_NB = _N // _R


def _mlp_body(x_ref, a_ref, w1_ref, b1_ref, w2_ref, b2_ref,
              p_ref, st_ref, acc):
    i = pl.program_id(0)
    z = x_ref[...] + a_ref[0] + a_ref[1]
    t = jnp.maximum(
        jnp.dot(z, w1_ref[...], preferred_element_type=jnp.float32)
        + b1_ref[...], 0.0)
    p = jnp.maximum(
        jnp.dot(t, w2_ref[...], preferred_element_type=jnp.float32)
        + b2_ref[...], 0.0)
    p_ref[...] = p

    @pl.when(i == 0)
    def _():
        acc[...] = jnp.zeros_like(acc)

    s = jnp.sum(p, axis=0, keepdims=True)
    ss = jnp.sum(p * p, axis=0, keepdims=True)
    acc[...] += jnp.concatenate([s, ss], axis=0)

    @pl.when(i == _NB - 1)
    def _():
        st_ref[...] = acc[...]


def _mlp(x, agg, W1, b1, W2, b2):
    return pl.pallas_call(
        _mlp_body,
        grid=(_NB,),
        in_specs=[
            pl.BlockSpec((_R, _D), lambda i: (i, 0)),
            pl.BlockSpec((_NC, _R, _D), lambda i: (0, i, 0)),
            pl.BlockSpec((_D, _D), lambda i: (0, 0)),
            pl.BlockSpec((1, _D), lambda i: (0, 0)),
            pl.BlockSpec((_D, _D), lambda i: (0, 0)),
            pl.BlockSpec((1, _D), lambda i: (0, 0)),
        ],
        out_specs=[
            pl.BlockSpec((_R, _D), lambda i: (i, 0)),
            pl.BlockSpec((2, _D), lambda i: (0, 0)),
        ],
        out_shape=[
            jax.ShapeDtypeStruct((_N, _D), jnp.float32),
            jax.ShapeDtypeStruct((2, _D), jnp.float32),
        ],
        scratch_shapes=[pltpu.VMEM((2, _D), jnp.float32)],
    )(x, agg, W1, b1, W2, b2)


def _norm_body(p_ref, st_ref, g_ref, be_ref, o_ref):
    mu = st_ref[0:1, :] / _N
    var = st_ref[1:2, :] / _N - mu * mu
    inv = jax.lax.rsqrt(var + 1e-5)
    o_ref[...] = (p_ref[...] - mu) * inv * g_ref[...] + be_ref[...]


def _norm(p, st, g, be):
    return pl.pallas_call(
        _norm_body,
        grid=(_NB,),
        in_specs=[
            pl.BlockSpec((_R, _D), lambda i: (i, 0)),
            pl.BlockSpec((2, _D), lambda i: (0, 0)),
            pl.BlockSpec((1, _D), lambda i: (0, 0)),
            pl.BlockSpec((1, _D), lambda i: (0, 0)),
        ],
        out_specs=pl.BlockSpec((_R, _D), lambda i: (i, 0)),
        out_shape=jax.ShapeDtypeStruct((_N, _D), jnp.float32),
    )(p, st, g, be)


def _final_body(x_ref, a_ref, w1_ref, b1_ref, w2_ref, b2_ref,
                wf_ref, bf_ref, o_ref):
    z = x_ref[...] + a_ref[0] + a_ref[1]
    t = jnp.maximum(
        jnp.dot(z, w1_ref[...], preferred_element_type=jnp.float32)
        + b1_ref[...], 0.0)
    h = jnp.maximum(
        jnp.dot(t, w2_ref[...], preferred_element_type=jnp.float32)
        + b2_ref[...], 0.0)
    o = jnp.dot(h, wf_ref[...], preferred_element_type=jnp.float32) + bf_ref[...]
    m = jnp.max(o, axis=-1, keepdims=True)
    lse = jnp.log(jnp.sum(jnp.exp(o - m), axis=-1, keepdims=True)) + m
    o_ref[...] = o - lse


def _final(x, agg, W1, b1, W2, b2, Wf, bf):
    return pl.pallas_call(
        _final_body,
        grid=(_NB,),
        in_specs=[
            pl.BlockSpec((_R, _D), lambda i: (i, 0)),
            pl.BlockSpec((_NC, _R, _D), lambda i: (0, i, 0)),
            pl.BlockSpec((_D, _D), lambda i: (0, 0)),
            pl.BlockSpec((1, _D), lambda i: (0, 0)),
            pl.BlockSpec((_D, _D), lambda i: (0, 0)),
            pl.BlockSpec((1, _D), lambda i: (0, 0)),
            pl.BlockSpec((_D, _C), lambda i: (0, 0)),
            pl.BlockSpec((1, _C), lambda i: (0, 0)),
        ],
        out_specs=pl.BlockSpec((_R, _C), lambda i: (i, 0)),
        out_shape=jax.ShapeDtypeStruct((_N, _C), jnp.float32),
    )(x, agg, W1, b1, W2, b2, Wf, bf)


def kernel(x, edge_index, W11, b11, W12, b12, W21, b21, W22, b22,
           W31, b31, W32, b32, g1, be1, g2, be2, Wf, bf):
    edges = _split_edges(edge_index)
    zeros = jnp.zeros((_RPT, _D), jnp.float32)

    b11r, b12r = b11.reshape(1, _D), b12.reshape(1, _D)
    b21r, b22r = b21.reshape(1, _D), b22.reshape(1, _D)
    b31r, b32r = b31.reshape(1, _D), b32.reshape(1, _D)
    bfr = bf.reshape(1, _C)
    g1r, be1r = g1.reshape(1, _D), be1.reshape(1, _D)
    g2r, be2r = g2.reshape(1, _D), be2.reshape(1, _D)

    agg1 = _sc_agg(x, edges, zeros)
    p1, st1 = _mlp(x, agg1, W11, b11r, W12, b12r)
    h1 = _norm(p1, st1, g1r, be1r)
    agg2 = _sc_agg(h1, edges, zeros)
    p2, st2 = _mlp(h1, agg2, W21, b21r, W22, b22r)
    h2 = _norm(p2, st2, g2r, be2r)
    agg3 = _sc_agg(h2, edges, zeros)
    return _final(h2, agg3, W31, b31r, W32, b32r, Wf, bfr)


# speedup vs baseline: 1.6273x; 1.6273x over previous
import functools

import jax
import jax.numpy as jnp
from jax import lax
from jax.experimental import pallas as pl
from jax.experimental.pallas import tpu as pltpu
from jax.experimental.pallas import tpu_sc as plsc

_N, _D, _C, _E = 10000, 128, 16, 320000


_NC, _NS = 2, 16
_NW = _NC * _NS
_EPW = _E // _NW
_CH = 128
_FULL = _EPW // _CH
_TAIL = _EPW - _FULL * _CH
_RPT = 624
_RTAIL = _N - _RPT * _NS

@functools.cache
def _make_sc_agg():
    mesh = plsc.VectorSubcoreMesh(core_axis_name="c", subcore_axis_name="s")
    return pl.kernel(
        _sc_agg_body,
        mesh=mesh,
        out_type=jax.ShapeDtypeStruct((_NC, _N, _D), jnp.float32),
        scratch_types=[
            pltpu.VMEM((_FULL * _CH,), jnp.int32),
            pltpu.VMEM((_CH,), jnp.int32),
            pltpu.VMEM((_CH,), jnp.int32),
            pltpu.VMEM((_CH, _D), jnp.float32),
            pltpu.VMEM((_CH, _D), jnp.float32),
            pltpu.VMEM((_TAIL,), jnp.int32),
            pltpu.VMEM((_TAIL,), jnp.int32),
            pltpu.VMEM((_TAIL, _D), jnp.float32),
            pltpu.VMEM_SHARED((_N, _D), jnp.float32),
            pltpu.SemaphoreType.DMA,
            pltpu.SemaphoreType.DMA,
            pltpu.SemaphoreType.DMA,
            pltpu.SemaphoreType.DMA,
            pltpu.SemaphoreType.DMA,
            pltpu.SemaphoreType.DMA,
        ],
    )


def _sc_agg(x, src, dst, zeros):
    return _make_sc_agg()(x, src, dst, zeros)


def _sc_agg_body(x_hbm, src_hbm, dst_hbm, zero_hbm,
                 out_hbm, src_v, db0, db1, rows0, rows1, src_t, dst_t, rows_t,
                 agg_sh, sd0, sd1, sg0, sg1, ss0, ss1):
    cid = lax.axis_index("c")
    sid = lax.axis_index("s")
    wid = sid * _NC + cid
    db = (db0, db1)
    rows = (rows0, rows1)
    sd = (sd0, sd1)
    sg = (sg0, sg1)
    ss = (ss0, ss1)

    pltpu.sync_copy(zero_hbm.at[pl.ds(0, _RPT)],
                    agg_sh.at[pl.ds(sid * _RPT, _RPT)])

    @pl.when(sid == _NS - 1)
    def _():
        pltpu.sync_copy(zero_hbm.at[pl.ds(0, _RTAIL)],
                        agg_sh.at[pl.ds(_NS * _RPT, _RTAIL)])

    e0 = wid * _EPW
    pltpu.sync_copy(src_hbm.at[pl.ds(e0, _FULL * _CH)], src_v)
    plsc.subcore_barrier()

    def _dst_load(b, i):
        pltpu.async_copy(dst_hbm.at[pl.ds(e0 + i * _CH, _CH)], db[b], sd[b])

    def _wait_dst(b, i):
        pltpu.make_async_copy(
            dst_hbm.at[pl.ds(e0 + i * _CH, _CH)], db[b], sd[b]).wait()

    def _gather(b, i):
        pltpu.async_copy(x_hbm.at[src_v.at[pl.ds(i * _CH, _CH)]],
                         rows[b], sg[b])

    def _wait_gather(b, i):
        pltpu.make_async_copy(x_hbm.at[src_v.at[pl.ds(i * _CH, _CH)]],
                              rows[b], sg[b]).wait()

    def _scatter(b):
        pltpu.async_copy(rows[b], agg_sh.at[db[b]], ss[b], add=True)

    def _wait_scatter(b):
        pltpu.make_async_copy(rows[b], agg_sh.at[db[b]], ss[b]).wait()

    _dst_load(0, 0)
    _gather(0, 0)

    def body(j, carry):
        i0 = 2 * j
        i1 = 2 * j + 1

        @pl.when(j > 0)
        def _():
            _wait_scatter(1)

        _dst_load(1, i1)
        _gather(1, i1)
        _wait_gather(0, i0)
        _wait_dst(0, i0)
        _scatter(0)

        @pl.when(j < _FULL // 2 - 1)
        def _():
            _wait_scatter(0)
            _dst_load(0, i0 + 2)
            _gather(0, i0 + 2)

        _wait_gather(1, i1)
        _wait_dst(1, i1)
        _scatter(1)
        return carry

    lax.fori_loop(0, _FULL // 2, body, 0)
    _wait_scatter(0)
    _wait_scatter(1)

    pltpu.sync_copy(src_hbm.at[pl.ds(e0 + _FULL * _CH, _TAIL)], src_t)
    pltpu.sync_copy(dst_hbm.at[pl.ds(e0 + _FULL * _CH, _TAIL)], dst_t)
    pltpu.async_copy(x_hbm.at[src_t], rows_t, sg0).wait()
    pltpu.sync_copy(rows_t, agg_sh.at[dst_t], add=True)

    plsc.subcore_barrier()
    pltpu.sync_copy(agg_sh.at[pl.ds(sid * _RPT, _RPT)],
                    out_hbm.at[cid, pl.ds(sid * _RPT, _RPT)])

    @pl.when(sid == _NS - 1)
    def _():
        pltpu.sync_copy(agg_sh.at[pl.ds(_NS * _RPT, _RTAIL)],
                        out_hbm.at[cid, pl.ds(_NS * _RPT, _RTAIL)])


_R = 2000
_NB = _N // _R


def _mlp_body(x_ref, a_ref, w1_ref, b1_ref, w2_ref, b2_ref,
              p_ref, st_ref, acc):
    i = pl.program_id(0)
    z = x_ref[...] + a_ref[0] + a_ref[1]
    t = jnp.maximum(
        jnp.dot(z, w1_ref[...], preferred_element_type=jnp.float32)
        + b1_ref[...], 0.0)
    p = jnp.maximum(
        jnp.dot(t, w2_ref[...], preferred_element_type=jnp.float32)
        + b2_ref[...], 0.0)
    p_ref[...] = p

    @pl.when(i == 0)
    def _():
        acc[...] = jnp.zeros_like(acc)

    s = jnp.sum(p, axis=0, keepdims=True)
    ss = jnp.sum(p * p, axis=0, keepdims=True)
    acc[...] += jnp.concatenate([s, ss], axis=0)

    @pl.when(i == _NB - 1)
    def _():
        st_ref[...] = acc[...]


def _mlp(x, agg, W1, b1, W2, b2):
    return pl.pallas_call(
        _mlp_body,
        grid=(_NB,),
        in_specs=[
            pl.BlockSpec((_R, _D), lambda i: (i, 0)),
            pl.BlockSpec((_NC, _R, _D), lambda i: (0, i, 0)),
            pl.BlockSpec((_D, _D), lambda i: (0, 0)),
            pl.BlockSpec((1, _D), lambda i: (0, 0)),
            pl.BlockSpec((_D, _D), lambda i: (0, 0)),
            pl.BlockSpec((1, _D), lambda i: (0, 0)),
        ],
        out_specs=[
            pl.BlockSpec((_R, _D), lambda i: (i, 0)),
            pl.BlockSpec((2, _D), lambda i: (0, 0)),
        ],
        out_shape=[
            jax.ShapeDtypeStruct((_N, _D), jnp.float32),
            jax.ShapeDtypeStruct((2, _D), jnp.float32),
        ],
        scratch_shapes=[pltpu.VMEM((2, _D), jnp.float32)],
    )(x, agg, W1, b1, W2, b2)


def _norm_body(p_ref, st_ref, g_ref, be_ref, o_ref):
    mu = st_ref[0:1, :] / _N
    var = st_ref[1:2, :] / _N - mu * mu
    inv = jax.lax.rsqrt(var + 1e-5)
    o_ref[...] = (p_ref[...] - mu) * inv * g_ref[...] + be_ref[...]


def _norm(p, st, g, be):
    return pl.pallas_call(
        _norm_body,
        grid=(_NB,),
        in_specs=[
            pl.BlockSpec((_R, _D), lambda i: (i, 0)),
            pl.BlockSpec((2, _D), lambda i: (0, 0)),
            pl.BlockSpec((1, _D), lambda i: (0, 0)),
            pl.BlockSpec((1, _D), lambda i: (0, 0)),
        ],
        out_specs=pl.BlockSpec((_R, _D), lambda i: (i, 0)),
        out_shape=jax.ShapeDtypeStruct((_N, _D), jnp.float32),
    )(p, st, g, be)


def _final_body(x_ref, a_ref, w1_ref, b1_ref, w2_ref, b2_ref,
                wf_ref, bf_ref, o_ref):
    z = x_ref[...] + a_ref[0] + a_ref[1]
    t = jnp.maximum(
        jnp.dot(z, w1_ref[...], preferred_element_type=jnp.float32)
        + b1_ref[...], 0.0)
    h = jnp.maximum(
        jnp.dot(t, w2_ref[...], preferred_element_type=jnp.float32)
        + b2_ref[...], 0.0)
    o = jnp.dot(h, wf_ref[...], preferred_element_type=jnp.float32) + bf_ref[...]
    m = jnp.max(o, axis=-1, keepdims=True)
    lse = jnp.log(jnp.sum(jnp.exp(o - m), axis=-1, keepdims=True)) + m
    o_ref[...] = o - lse


def _final(x, agg, W1, b1, W2, b2, Wf, bf):
    return pl.pallas_call(
        _final_body,
        grid=(_NB,),
        in_specs=[
            pl.BlockSpec((_R, _D), lambda i: (i, 0)),
            pl.BlockSpec((_NC, _R, _D), lambda i: (0, i, 0)),
            pl.BlockSpec((_D, _D), lambda i: (0, 0)),
            pl.BlockSpec((1, _D), lambda i: (0, 0)),
            pl.BlockSpec((_D, _D), lambda i: (0, 0)),
            pl.BlockSpec((1, _D), lambda i: (0, 0)),
            pl.BlockSpec((_D, _C), lambda i: (0, 0)),
            pl.BlockSpec((1, _C), lambda i: (0, 0)),
        ],
        out_specs=pl.BlockSpec((_R, _C), lambda i: (i, 0)),
        out_shape=jax.ShapeDtypeStruct((_N, _C), jnp.float32),
    )(x, agg, W1, b1, W2, b2, Wf, bf)


def kernel(x, edge_index, W11, b11, W12, b12, W21, b21, W22, b22,
           W31, b31, W32, b32, g1, be1, g2, be2, Wf, bf):
    srcv = edge_index[0]
    dstv = edge_index[1]
    zeros = jnp.zeros((_RPT, _D), jnp.float32)

    b11r, b12r = b11.reshape(1, _D), b12.reshape(1, _D)
    b21r, b22r = b21.reshape(1, _D), b22.reshape(1, _D)
    b31r, b32r = b31.reshape(1, _D), b32.reshape(1, _D)
    bfr = bf.reshape(1, _C)
    g1r, be1r = g1.reshape(1, _D), be1.reshape(1, _D)
    g2r, be2r = g2.reshape(1, _D), be2.reshape(1, _D)

    agg1 = _sc_agg(x, srcv, dstv, zeros)
    p1, st1 = _mlp(x, agg1, W11, b11r, W12, b12r)
    h1 = _norm(p1, st1, g1r, be1r)
    agg2 = _sc_agg(h1, srcv, dstv, zeros)
    p2, st2 = _mlp(h1, agg2, W21, b21r, W22, b22r)
    h2 = _norm(p2, st2, g2r, be2r)
    agg3 = _sc_agg(h2, srcv, dstv, zeros)
    return _final(h2, agg3, W31, b31r, W32, b32r, Wf, bfr)


# speedup vs baseline: 1.6753x; 1.0295x over previous
import functools

import jax
import jax.numpy as jnp
from jax import lax
from jax.experimental import pallas as pl
from jax.experimental.pallas import tpu as pltpu
from jax.experimental.pallas import tpu_sc as plsc

_N, _D, _C, _E = 10000, 128, 16, 320000


_NC, _NS = 2, 16
_NW = _NC * _NS
_EPW = _E // _NW
_CH = 128
_FULL = _EPW // _CH
_TAIL = _EPW - _FULL * _CH
_RPT = 624
_RTAIL = _N - _RPT * _NS

@functools.cache
def _make_sc_agg():
    mesh = plsc.VectorSubcoreMesh(core_axis_name="c", subcore_axis_name="s")
    return pl.kernel(
        _sc_agg_body,
        mesh=mesh,
        out_type=jax.ShapeDtypeStruct((_NC, _N, _D), jnp.float32),
        scratch_types=[
            pltpu.VMEM((_FULL * _CH,), jnp.int32),
            pltpu.VMEM((_CH,), jnp.int32),
            pltpu.VMEM((_CH,), jnp.int32),
            pltpu.VMEM((_CH, _D), jnp.float32),
            pltpu.VMEM((_CH, _D), jnp.float32),
            pltpu.VMEM((_TAIL,), jnp.int32),
            pltpu.VMEM((_TAIL,), jnp.int32),
            pltpu.VMEM((_TAIL, _D), jnp.float32),
            pltpu.VMEM_SHARED((_N, _D), jnp.float32),
            pltpu.SemaphoreType.DMA,
            pltpu.SemaphoreType.DMA,
            pltpu.SemaphoreType.DMA,
            pltpu.SemaphoreType.DMA,
            pltpu.SemaphoreType.DMA,
            pltpu.SemaphoreType.DMA,
        ],
    )


def _sc_agg(x, src, dst, zeros):
    return _make_sc_agg()(x, src, dst, zeros)


def _sc_agg_body(x_hbm, src_hbm, dst_hbm, zero_hbm,
                 out_hbm, src_v, db0, db1, rows0, rows1, src_t, dst_t, rows_t,
                 agg_sh, sd0, sd1, sg0, sg1, ss0, ss1):
    cid = lax.axis_index("c")
    sid = lax.axis_index("s")
    wid = sid * _NC + cid
    db = (db0, db1)
    rows = (rows0, rows1)
    sd = (sd0, sd1)
    sg = (sg0, sg1)
    ss = (ss0, ss1)

    pltpu.sync_copy(zero_hbm.at[pl.ds(0, _RPT)],
                    agg_sh.at[pl.ds(sid * _RPT, _RPT)])

    @pl.when(sid == _NS - 1)
    def _():
        pltpu.sync_copy(zero_hbm.at[pl.ds(0, _RTAIL)],
                        agg_sh.at[pl.ds(_NS * _RPT, _RTAIL)])

    e0 = wid * _EPW
    pltpu.sync_copy(src_hbm.at[pl.ds(e0, _FULL * _CH)], src_v)
    plsc.subcore_barrier()

    def _dst_load(b, i):
        pltpu.async_copy(dst_hbm.at[pl.ds(e0 + i * _CH, _CH)], db[b], sd[b])

    def _wait_dst(b, i):
        pltpu.make_async_copy(
            dst_hbm.at[pl.ds(e0 + i * _CH, _CH)], db[b], sd[b]).wait()

    def _gather(b, i):
        pltpu.async_copy(x_hbm.at[src_v.at[pl.ds(i * _CH, _CH)]],
                         rows[b], sg[b])

    def _wait_gather(b, i):
        pltpu.make_async_copy(x_hbm.at[src_v.at[pl.ds(i * _CH, _CH)]],
                              rows[b], sg[b]).wait()

    def _scatter(b):
        pltpu.async_copy(rows[b], agg_sh.at[db[b]], ss[b], add=True)

    def _wait_scatter(b):
        pltpu.make_async_copy(rows[b], agg_sh.at[db[b]], ss[b]).wait()

    _dst_load(0, 0)
    _gather(0, 0)

    def body(j, carry):
        i0 = 2 * j
        i1 = 2 * j + 1

        @pl.when(j > 0)
        def _():
            _wait_scatter(1)

        _dst_load(1, i1)
        _gather(1, i1)
        _wait_gather(0, i0)
        _wait_dst(0, i0)
        _scatter(0)

        @pl.when(j < _FULL // 2 - 1)
        def _():
            _wait_scatter(0)
            _dst_load(0, i0 + 2)
            _gather(0, i0 + 2)

        _wait_gather(1, i1)
        _wait_dst(1, i1)
        _scatter(1)
        return carry

    lax.fori_loop(0, _FULL // 2, body, 0)
    _wait_scatter(0)
    _wait_scatter(1)

    pltpu.sync_copy(src_hbm.at[pl.ds(e0 + _FULL * _CH, _TAIL)], src_t)
    pltpu.sync_copy(dst_hbm.at[pl.ds(e0 + _FULL * _CH, _TAIL)], dst_t)
    pltpu.async_copy(x_hbm.at[src_t], rows_t, sg0).wait()
    pltpu.sync_copy(rows_t, agg_sh.at[dst_t], add=True)

    plsc.subcore_barrier()
    pltpu.sync_copy(agg_sh.at[pl.ds(sid * _RPT, _RPT)],
                    out_hbm.at[cid, pl.ds(sid * _RPT, _RPT)])

    @pl.when(sid == _NS - 1)
    def _():
        pltpu.sync_copy(agg_sh.at[pl.ds(_NS * _RPT, _RTAIL)],
                        out_hbm.at[cid, pl.ds(_NS * _RPT, _RTAIL)])


def _edges_body(ei_ref, s_ref, d_ref):
    s_ref[...] = ei_ref[0]
    d_ref[...] = ei_ref[1]


def _edges_split(edge_index):
    return pl.pallas_call(
        _edges_body,
        out_shape=[jax.ShapeDtypeStruct((_E,), jnp.int32),
                   jax.ShapeDtypeStruct((_E,), jnp.int32)],
    )(edge_index)


_R = 2000
_NB = _N // _R


def _mlp_body(x_ref, a_ref, w1_ref, b1_ref, w2_ref, b2_ref,
              p_ref, st_ref, acc):
    i = pl.program_id(0)
    z = x_ref[...] + a_ref[0] + a_ref[1]
    t = jnp.maximum(
        jnp.dot(z, w1_ref[...], preferred_element_type=jnp.float32)
        + b1_ref[...], 0.0)
    p = jnp.maximum(
        jnp.dot(t, w2_ref[...], preferred_element_type=jnp.float32)
        + b2_ref[...], 0.0)
    p_ref[...] = p

    @pl.when(i == 0)
    def _():
        acc[...] = jnp.zeros_like(acc)

    s = jnp.sum(p, axis=0, keepdims=True)
    ss = jnp.sum(p * p, axis=0, keepdims=True)
    acc[...] += jnp.concatenate([s, ss], axis=0)

    @pl.when(i == _NB - 1)
    def _():
        st_ref[...] = acc[...]


def _mlp(x, agg, W1, b1, W2, b2):
    return pl.pallas_call(
        _mlp_body,
        grid=(_NB,),
        in_specs=[
            pl.BlockSpec((_R, _D), lambda i: (i, 0)),
            pl.BlockSpec((_NC, _R, _D), lambda i: (0, i, 0)),
            pl.BlockSpec((_D, _D), lambda i: (0, 0)),
            pl.BlockSpec((1, _D), lambda i: (0, 0)),
            pl.BlockSpec((_D, _D), lambda i: (0, 0)),
            pl.BlockSpec((1, _D), lambda i: (0, 0)),
        ],
        out_specs=[
            pl.BlockSpec((_R, _D), lambda i: (i, 0)),
            pl.BlockSpec((2, _D), lambda i: (0, 0)),
        ],
        out_shape=[
            jax.ShapeDtypeStruct((_N, _D), jnp.float32),
            jax.ShapeDtypeStruct((2, _D), jnp.float32),
        ],
        scratch_shapes=[pltpu.VMEM((2, _D), jnp.float32)],
    )(x, agg, W1, b1, W2, b2)


def _norm_body(p_ref, st_ref, g_ref, be_ref, o_ref):
    mu = st_ref[0:1, :] / _N
    var = st_ref[1:2, :] / _N - mu * mu
    inv = jax.lax.rsqrt(var + 1e-5)
    o_ref[...] = (p_ref[...] - mu) * inv * g_ref[...] + be_ref[...]


def _norm(p, st, g, be):
    return pl.pallas_call(
        _norm_body,
        grid=(_NB,),
        in_specs=[
            pl.BlockSpec((_R, _D), lambda i: (i, 0)),
            pl.BlockSpec((2, _D), lambda i: (0, 0)),
            pl.BlockSpec((1, _D), lambda i: (0, 0)),
            pl.BlockSpec((1, _D), lambda i: (0, 0)),
        ],
        out_specs=pl.BlockSpec((_R, _D), lambda i: (i, 0)),
        out_shape=jax.ShapeDtypeStruct((_N, _D), jnp.float32),
    )(p, st, g, be)


def _final_body(x_ref, a_ref, w1_ref, b1_ref, w2_ref, b2_ref,
                wf_ref, bf_ref, o_ref):
    z = x_ref[...] + a_ref[0] + a_ref[1]
    t = jnp.maximum(
        jnp.dot(z, w1_ref[...], preferred_element_type=jnp.float32)
        + b1_ref[...], 0.0)
    h = jnp.maximum(
        jnp.dot(t, w2_ref[...], preferred_element_type=jnp.float32)
        + b2_ref[...], 0.0)
    o = jnp.dot(h, wf_ref[...], preferred_element_type=jnp.float32) + bf_ref[...]
    m = jnp.max(o, axis=-1, keepdims=True)
    lse = jnp.log(jnp.sum(jnp.exp(o - m), axis=-1, keepdims=True)) + m
    o_ref[...] = o - lse


def _final(x, agg, W1, b1, W2, b2, Wf, bf):
    return pl.pallas_call(
        _final_body,
        grid=(_NB,),
        in_specs=[
            pl.BlockSpec((_R, _D), lambda i: (i, 0)),
            pl.BlockSpec((_NC, _R, _D), lambda i: (0, i, 0)),
            pl.BlockSpec((_D, _D), lambda i: (0, 0)),
            pl.BlockSpec((1, _D), lambda i: (0, 0)),
            pl.BlockSpec((_D, _D), lambda i: (0, 0)),
            pl.BlockSpec((1, _D), lambda i: (0, 0)),
            pl.BlockSpec((_D, _C), lambda i: (0, 0)),
            pl.BlockSpec((1, _C), lambda i: (0, 0)),
        ],
        out_specs=pl.BlockSpec((_R, _C), lambda i: (i, 0)),
        out_shape=jax.ShapeDtypeStruct((_N, _C), jnp.float32),
    )(x, agg, W1, b1, W2, b2, Wf, bf)


def kernel(x, edge_index, W11, b11, W12, b12, W21, b21, W22, b22,
           W31, b31, W32, b32, g1, be1, g2, be2, Wf, bf):
    srcv, dstv = _edges_split(edge_index)
    zeros = jnp.zeros((_RPT, _D), jnp.float32)

    b11r, b12r = b11.reshape(1, _D), b12.reshape(1, _D)
    b21r, b22r = b21.reshape(1, _D), b22.reshape(1, _D)
    b31r, b32r = b31.reshape(1, _D), b32.reshape(1, _D)
    bfr = bf.reshape(1, _C)
    g1r, be1r = g1.reshape(1, _D), be1.reshape(1, _D)
    g2r, be2r = g2.reshape(1, _D), be2.reshape(1, _D)

    agg1 = _sc_agg(x, srcv, dstv, zeros)
    p1, st1 = _mlp(x, agg1, W11, b11r, W12, b12r)
    h1 = _norm(p1, st1, g1r, be1r)
    agg2 = _sc_agg(h1, srcv, dstv, zeros)
    p2, st2 = _mlp(h1, agg2, W21, b21r, W22, b22r)
    h2 = _norm(p2, st2, g2r, be2r)
    agg3 = _sc_agg(h2, srcv, dstv, zeros)
    return _final(h2, agg3, W31, b31r, W32, b32r, Wf, bfr)


# speedup vs baseline: 1.7105x; 1.0210x over previous
import functools

import jax
import jax.numpy as jnp
from jax import lax
from jax.experimental import pallas as pl
from jax.experimental.pallas import tpu as pltpu
from jax.experimental.pallas import tpu_sc as plsc

_N, _D, _C, _E = 10000, 128, 16, 320000


_NC, _NS = 2, 16
_NW = _NC * _NS
_EPW = _E // _NW
_CH = 128
_FULL = _EPW // _CH
_TAIL = _EPW - _FULL * _CH
_RPT = 624
_RTAIL = _N - _RPT * _NS

@functools.cache
def _make_sc_agg():
    mesh = plsc.VectorSubcoreMesh(core_axis_name="c", subcore_axis_name="s")
    return pl.kernel(
        _sc_agg_body,
        mesh=mesh,
        out_type=jax.ShapeDtypeStruct((_NC, _N, _D), jnp.float32),
        scratch_types=[
            pltpu.VMEM((_FULL * _CH,), jnp.int32),
            pltpu.VMEM((_CH,), jnp.int32),
            pltpu.VMEM((_CH,), jnp.int32),
            pltpu.VMEM((_CH, _D), jnp.float32),
            pltpu.VMEM((_CH, _D), jnp.float32),
            pltpu.VMEM((_TAIL,), jnp.int32),
            pltpu.VMEM((_TAIL,), jnp.int32),
            pltpu.VMEM((_TAIL, _D), jnp.float32),
            pltpu.VMEM_SHARED((_N, _D), jnp.float32),
            pltpu.SemaphoreType.DMA,
            pltpu.SemaphoreType.DMA,
            pltpu.SemaphoreType.DMA,
            pltpu.SemaphoreType.DMA,
            pltpu.SemaphoreType.DMA,
            pltpu.SemaphoreType.DMA,
        ],
    )


def _sc_agg(x, src, dst, zeros):
    return _make_sc_agg()(x, src, dst, zeros)


def _sc_agg_body(x_hbm, src_hbm, dst_hbm, zero_hbm,
                 out_hbm, src_v, db0, db1, rows0, rows1, src_t, dst_t, rows_t,
                 agg_sh, sd0, sd1, sg0, sg1, ss0, ss1):
    cid = lax.axis_index("c")
    sid = lax.axis_index("s")
    wid = sid * _NC + cid
    db = (db0, db1)
    rows = (rows0, rows1)
    sd = (sd0, sd1)
    sg = (sg0, sg1)
    ss = (ss0, ss1)

    pltpu.sync_copy(zero_hbm.at[pl.ds(0, _RPT)],
                    agg_sh.at[pl.ds(sid * _RPT, _RPT)])

    @pl.when(sid == _NS - 1)
    def _():
        pltpu.sync_copy(zero_hbm.at[pl.ds(0, _RTAIL)],
                        agg_sh.at[pl.ds(_NS * _RPT, _RTAIL)])

    e0 = wid * _EPW
    pltpu.sync_copy(src_hbm.at[pl.ds(e0, _FULL * _CH)], src_v)
    plsc.subcore_barrier()

    def _dst_load(b, i):
        pltpu.async_copy(dst_hbm.at[pl.ds(e0 + i * _CH, _CH)], db[b], sd[b])

    def _wait_dst(b, i):
        pltpu.make_async_copy(
            dst_hbm.at[pl.ds(e0 + i * _CH, _CH)], db[b], sd[b]).wait()

    def _gather(b, i):
        pltpu.async_copy(x_hbm.at[src_v.at[pl.ds(i * _CH, _CH)]],
                         rows[b], sg[b])

    def _wait_gather(b, i):
        pltpu.make_async_copy(x_hbm.at[src_v.at[pl.ds(i * _CH, _CH)]],
                              rows[b], sg[b]).wait()

    def _scatter(b):
        pltpu.async_copy(rows[b], agg_sh.at[db[b]], ss[b], add=True)

    def _wait_scatter(b):
        pltpu.make_async_copy(rows[b], agg_sh.at[db[b]], ss[b]).wait()

    _dst_load(0, 0)
    _gather(0, 0)

    def body(j, carry):
        i0 = 2 * j
        i1 = 2 * j + 1

        @pl.when(j > 0)
        def _():
            _wait_scatter(1)

        _dst_load(1, i1)
        _gather(1, i1)
        _wait_gather(0, i0)
        _wait_dst(0, i0)
        _scatter(0)

        @pl.when(j < _FULL // 2 - 1)
        def _():
            _wait_scatter(0)
            _dst_load(0, i0 + 2)
            _gather(0, i0 + 2)

        _wait_gather(1, i1)
        _wait_dst(1, i1)
        _scatter(1)
        return carry

    lax.fori_loop(0, _FULL // 2, body, 0)
    _wait_scatter(0)
    _wait_scatter(1)

    pltpu.sync_copy(src_hbm.at[pl.ds(e0 + _FULL * _CH, _TAIL)], src_t)
    pltpu.sync_copy(dst_hbm.at[pl.ds(e0 + _FULL * _CH, _TAIL)], dst_t)
    pltpu.async_copy(x_hbm.at[src_t], rows_t, sg0).wait()
    pltpu.sync_copy(rows_t, agg_sh.at[dst_t], add=True)

    plsc.subcore_barrier()
    pltpu.sync_copy(agg_sh.at[pl.ds(sid * _RPT, _RPT)],
                    out_hbm.at[cid, pl.ds(sid * _RPT, _RPT)])

    @pl.when(sid == _NS - 1)
    def _():
        pltpu.sync_copy(agg_sh.at[pl.ds(_NS * _RPT, _RTAIL)],
                        out_hbm.at[cid, pl.ds(_NS * _RPT, _RTAIL)])


def _edges_body(ei_ref, s_ref, d_ref):
    s_ref[...] = ei_ref[0]
    d_ref[...] = ei_ref[1]


def _edges_split(edge_index):
    return pl.pallas_call(
        _edges_body,
        out_shape=[jax.ShapeDtypeStruct((_E,), jnp.int32),
                   jax.ShapeDtypeStruct((_E,), jnp.int32)],
    )(edge_index)


_R = 5000
_NB = _N // _R


def _mlp_body(x_ref, a_ref, w1_ref, b1_ref, w2_ref, b2_ref,
              p_ref, st_ref, acc):
    i = pl.program_id(0)
    z = x_ref[...] + a_ref[0] + a_ref[1]
    t = jnp.maximum(
        jnp.dot(z, w1_ref[...], preferred_element_type=jnp.float32)
        + b1_ref[...], 0.0)
    p = jnp.maximum(
        jnp.dot(t, w2_ref[...], preferred_element_type=jnp.float32)
        + b2_ref[...], 0.0)
    p_ref[...] = p

    @pl.when(i == 0)
    def _():
        acc[...] = jnp.zeros_like(acc)

    s = jnp.sum(p, axis=0, keepdims=True)
    ss = jnp.sum(p * p, axis=0, keepdims=True)
    acc[...] += jnp.concatenate([s, ss], axis=0)

    @pl.when(i == _NB - 1)
    def _():
        st_ref[...] = acc[...]


def _mlp(x, agg, W1, b1, W2, b2):
    return pl.pallas_call(
        _mlp_body,
        grid=(_NB,),
        in_specs=[
            pl.BlockSpec((_R, _D), lambda i: (i, 0)),
            pl.BlockSpec((_NC, _R, _D), lambda i: (0, i, 0)),
            pl.BlockSpec((_D, _D), lambda i: (0, 0)),
            pl.BlockSpec((1, _D), lambda i: (0, 0)),
            pl.BlockSpec((_D, _D), lambda i: (0, 0)),
            pl.BlockSpec((1, _D), lambda i: (0, 0)),
        ],
        out_specs=[
            pl.BlockSpec((_R, _D), lambda i: (i, 0)),
            pl.BlockSpec((2, _D), lambda i: (0, 0)),
        ],
        out_shape=[
            jax.ShapeDtypeStruct((_N, _D), jnp.float32),
            jax.ShapeDtypeStruct((2, _D), jnp.float32),
        ],
        scratch_shapes=[pltpu.VMEM((2, _D), jnp.float32)],
    )(x, agg, W1, b1, W2, b2)


def _norm_body(p_ref, st_ref, g_ref, be_ref, o_ref):
    mu = st_ref[0:1, :] / _N
    var = st_ref[1:2, :] / _N - mu * mu
    inv = jax.lax.rsqrt(var + 1e-5)
    o_ref[...] = (p_ref[...] - mu) * inv * g_ref[...] + be_ref[...]


def _norm(p, st, g, be):
    return pl.pallas_call(
        _norm_body,
        grid=(_NB,),
        in_specs=[
            pl.BlockSpec((_R, _D), lambda i: (i, 0)),
            pl.BlockSpec((2, _D), lambda i: (0, 0)),
            pl.BlockSpec((1, _D), lambda i: (0, 0)),
            pl.BlockSpec((1, _D), lambda i: (0, 0)),
        ],
        out_specs=pl.BlockSpec((_R, _D), lambda i: (i, 0)),
        out_shape=jax.ShapeDtypeStruct((_N, _D), jnp.float32),
    )(p, st, g, be)


def _final_body(x_ref, a_ref, w1_ref, b1_ref, w2_ref, b2_ref,
                wf_ref, bf_ref, o_ref):
    z = x_ref[...] + a_ref[0] + a_ref[1]
    t = jnp.maximum(
        jnp.dot(z, w1_ref[...], preferred_element_type=jnp.float32)
        + b1_ref[...], 0.0)
    h = jnp.maximum(
        jnp.dot(t, w2_ref[...], preferred_element_type=jnp.float32)
        + b2_ref[...], 0.0)
    o = jnp.dot(h, wf_ref[...], preferred_element_type=jnp.float32) + bf_ref[...]
    m = jnp.max(o, axis=-1, keepdims=True)
    lse = jnp.log(jnp.sum(jnp.exp(o - m), axis=-1, keepdims=True)) + m
    o_ref[...] = o - lse


def _final(x, agg, W1, b1, W2, b2, Wf, bf):
    return pl.pallas_call(
        _final_body,
        grid=(_NB,),
        in_specs=[
            pl.BlockSpec((_R, _D), lambda i: (i, 0)),
            pl.BlockSpec((_NC, _R, _D), lambda i: (0, i, 0)),
            pl.BlockSpec((_D, _D), lambda i: (0, 0)),
            pl.BlockSpec((1, _D), lambda i: (0, 0)),
            pl.BlockSpec((_D, _D), lambda i: (0, 0)),
            pl.BlockSpec((1, _D), lambda i: (0, 0)),
            pl.BlockSpec((_D, _C), lambda i: (0, 0)),
            pl.BlockSpec((1, _C), lambda i: (0, 0)),
        ],
        out_specs=pl.BlockSpec((_R, _C), lambda i: (i, 0)),
        out_shape=jax.ShapeDtypeStruct((_N, _C), jnp.float32),
    )(x, agg, W1, b1, W2, b2, Wf, bf)


def kernel(x, edge_index, W11, b11, W12, b12, W21, b21, W22, b22,
           W31, b31, W32, b32, g1, be1, g2, be2, Wf, bf):
    srcv, dstv = _edges_split(edge_index)
    zeros = jnp.zeros((_RPT, _D), jnp.float32)

    b11r, b12r = b11.reshape(1, _D), b12.reshape(1, _D)
    b21r, b22r = b21.reshape(1, _D), b22.reshape(1, _D)
    b31r, b32r = b31.reshape(1, _D), b32.reshape(1, _D)
    bfr = bf.reshape(1, _C)
    g1r, be1r = g1.reshape(1, _D), be1.reshape(1, _D)
    g2r, be2r = g2.reshape(1, _D), be2.reshape(1, _D)

    agg1 = _sc_agg(x, srcv, dstv, zeros)
    p1, st1 = _mlp(x, agg1, W11, b11r, W12, b12r)
    h1 = _norm(p1, st1, g1r, be1r)
    agg2 = _sc_agg(h1, srcv, dstv, zeros)
    p2, st2 = _mlp(h1, agg2, W21, b21r, W22, b22r)
    h2 = _norm(p2, st2, g2r, be2r)
    agg3 = _sc_agg(h2, srcv, dstv, zeros)
    return _final(h2, agg3, W31, b31r, W32, b32r, Wf, bfr)
